# baseline (speedup 1.0000x reference)
import functools

import jax
import jax.numpy as jnp
from jax import lax
from jax.experimental import pallas as pl
from jax.experimental.pallas import tpu as pltpu
from jax.experimental.pallas import tpu_sc as plsc

B = 16384
D = 64
NUM_NEG = 4
NC = 2
NS = 16
NW = NC * NS
NSEG = 2 + NUM_NEG


_mesh = plsc.VectorSubcoreMesh(core_axis_name="c", subcore_axis_name="s")
_params = pltpu.CompilerParams(use_tc_tiling_on_sc=False)


def _make_gather(nrows, nbuf, chunk):
    nchunk = nrows // (NW * chunk)
    assert nchunk * NW * chunk == nrows

    @functools.partial(
        pl.kernel,
        mesh=_mesh,
        out_type=jax.ShapeDtypeStruct((nrows, D), jnp.float32),
        scratch_types=(
            [pltpu.VMEM((chunk,), jnp.int32) for _ in range(nchunk)]
            + [pltpu.VMEM((chunk, D), jnp.float32) for _ in range(nbuf)]
            + [pltpu.SemaphoreType.DMA for _ in range(2 * nbuf + 1)]
        ),
        compiler_params=_params,
    )
    def gather(table, ids, out, *refs):
        idxs = refs[:nchunk]
        bufs = refs[nchunk:nchunk + nbuf]
        gsem = refs[nchunk + nbuf:nchunk + 2 * nbuf]
        ssem = refs[nchunk + 2 * nbuf:nchunk + 3 * nbuf]
        isem = refs[nchunk + 3 * nbuf]
        wid = lax.axis_index("s") * NC + lax.axis_index("c")
        base = wid * chunk

        idx_copies = [
            pltpu.async_copy(ids.at[pl.ds(j * NW * chunk + base, chunk)], idxs[j], isem)
            for j in range(nchunk)
        ]
        for c in idx_copies:
            c.wait()

        gathers = [None] * nchunk
        stores = [None] * nchunk

        def start_gather(k):
            gathers[k] = pltpu.async_copy(
                table.at[idxs[k]], bufs[k % nbuf], gsem[k % nbuf])

        def start_store(k):
            stores[k] = pltpu.async_copy(
                bufs[k % nbuf], out.at[pl.ds(k * NW * chunk + base, chunk)],
                ssem[k % nbuf])

        for k in range(min(nbuf, nchunk)):
            start_gather(k)
        for k in range(nchunk):
            gathers[k].wait()
            start_store(k)
            nk = k + nbuf
            if nk < nchunk:
                stores[nk - nbuf].wait()
                start_gather(nk)
        for k in range(max(0, nchunk - nbuf), nchunk):
            stores[k].wait()

    return gather


_gather_user = _make_gather(B, 1, B // NW)
_gather_item = _make_gather((NSEG - 1) * B, 3, B // NW)


def kernel(user_ids, item_ids, ne_item_ids, user_weight, item_weight):
    item_idx = jnp.concatenate([item_ids, ne_item_ids.reshape(-1)])
    user_emb = _gather_user(user_weight, user_ids)
    item_emb = _gather_item(item_weight, item_idx)
    return jnp.concatenate([user_emb, item_emb], axis=0)
